# baseline (speedup 1.0000x reference)
import functools

import jax
import jax.numpy as jnp
from jax import lax
from jax.experimental import pallas as pl
from jax.experimental.pallas import tpu as pltpu
from jax.experimental.pallas import tpu_sc as plsc

EMB_DIM = 64
SCALE = 8.0

NUM_CORES = 2
NUM_SUBCORES = 16
NUM_WORKERS = NUM_CORES * NUM_SUBCORES

CHUNK = 128


def _make_gather(B):
    assert B % (NUM_WORKERS * CHUNK) == 0
    b_per_w = B // NUM_WORKERS
    n_chunks = b_per_w // CHUNK
    mesh = plsc.VectorSubcoreMesh(core_axis_name="c", subcore_axis_name="s")

    @functools.partial(
        pl.kernel,
        mesh=mesh,
        out_type=jax.ShapeDtypeStruct((B, EMB_DIM), jnp.float32),
        scratch_types=[
            pltpu.VMEM((CHUNK,), jnp.int32),
            pltpu.VMEM((CHUNK, EMB_DIM), jnp.float32),
            pltpu.SemaphoreType.DMA,
        ],
        compiler_params=pltpu.CompilerParams(use_tc_tiling_on_sc=False),
    )
    def gather_kernel(table_hbm, idx_hbm, out_hbm, idx_v, rows_v, sem):
        wid = lax.axis_index("s") * NUM_CORES + lax.axis_index("c")
        base = wid * b_per_w

        def body(g, _):
            off = base + g * CHUNK
            pltpu.sync_copy(idx_hbm.at[pl.ds(off, CHUNK)], idx_v)
            pltpu.async_copy(table_hbm.at[idx_v], rows_v, sem).wait()

            def scale_body(i, _):
                for r in range(4):
                    row = rows_v.at[4 * i + r]
                    for j in range(EMB_DIM // 16):
                        row[pl.ds(16 * j, 16)] = row[pl.ds(16 * j, 16)] * SCALE
                return 0

            lax.fori_loop(0, CHUNK // 4, scale_body, 0, unroll=False)
            pltpu.sync_copy(rows_v, out_hbm.at[pl.ds(off, CHUNK)])
            return 0

        lax.fori_loop(0, n_chunks, body, 0, unroll=False)

    return gather_kernel


def kernel(x, table):
    S0, S1 = x.shape
    B = S0 * S1
    idx = x.reshape(B)
    out = _make_gather(B)(table, idx)
    return out.reshape(S0, S1, EMB_DIM)


# speedup vs baseline: 1.1306x; 1.1306x over previous
import functools

import jax
import jax.numpy as jnp
from jax import lax
from jax.experimental import pallas as pl
from jax.experimental.pallas import tpu as pltpu
from jax.experimental.pallas import tpu_sc as plsc

EMB_DIM = 64
SCALE = 8.0

NUM_CORES = 2
NUM_SUBCORES = 16
NUM_WORKERS = NUM_CORES * NUM_SUBCORES

CHUNK = 128
NBUF = 4


def _make_gather(B):
    assert B % (NUM_WORKERS * CHUNK * NBUF) == 0
    b_per_w = B // NUM_WORKERS
    n_chunks = b_per_w // CHUNK
    n_groups = n_chunks // NBUF
    mesh = plsc.VectorSubcoreMesh(core_axis_name="c", subcore_axis_name="s")

    @functools.partial(
        pl.kernel,
        mesh=mesh,
        out_type=jax.ShapeDtypeStruct((B, EMB_DIM), jnp.float32),
        scratch_types=[
            pltpu.VMEM((NBUF, CHUNK), jnp.int32),
            pltpu.VMEM((NBUF, CHUNK, EMB_DIM), jnp.float32),
            [pltpu.SemaphoreType.DMA] * NBUF,
            [pltpu.SemaphoreType.DMA] * NBUF,
        ],
        compiler_params=pltpu.CompilerParams(use_tc_tiling_on_sc=False),
    )
    def gather_kernel(table_hbm, idx_hbm, out_hbm, idx_v, rows_v, gsems, ssems):
        wid = lax.axis_index("s") * NUM_CORES + lax.axis_index("c")
        base = wid * b_per_w

        def start_gather(s, off):
            pltpu.sync_copy(idx_hbm.at[pl.ds(off, CHUNK)], idx_v.at[s])
            pltpu.make_async_copy(
                table_hbm.at[idx_v.at[s]], rows_v.at[s], gsems[s]
            ).start()

        def scale(s):
            def scale_body(i, _):
                for r in range(4):
                    row = rows_v.at[s].at[4 * i + r]
                    for j in range(EMB_DIM // 16):
                        row[pl.ds(16 * j, 16)] = row[pl.ds(16 * j, 16)] * SCALE
                return 0

            lax.fori_loop(0, CHUNK // 4, scale_body, 0, unroll=False)

        def process(s, off, prefetch):
            pltpu.make_async_copy(
                table_hbm.at[idx_v.at[s]], rows_v.at[s], gsems[s]
            ).wait()
            scale(s)
            pltpu.make_async_copy(
                rows_v.at[s], out_hbm.at[pl.ds(off, CHUNK)], ssems[s]
            ).start()
            pltpu.make_async_copy(
                rows_v.at[s], out_hbm.at[pl.ds(off, CHUNK)], ssems[s]
            ).wait()
            if prefetch:
                start_gather(s, off + NBUF * CHUNK)

        for s in range(NBUF):
            start_gather(s, base + s * CHUNK)

        def body(i, _):
            off = base + i * NBUF * CHUNK
            for s in range(NBUF):
                process(s, off + s * CHUNK, prefetch=True)
            return 0

        lax.fori_loop(0, n_groups - 1, body, 0, unroll=False)
        tail = base + (n_groups - 1) * NBUF * CHUNK
        for s in range(NBUF):
            process(s, tail + s * CHUNK, prefetch=False)

    return gather_kernel


def kernel(x, table):
    S0, S1 = x.shape
    B = S0 * S1
    idx = x.reshape(B)
    out = _make_gather(B)(table, idx)
    return out.reshape(S0, S1, EMB_DIM)
